# baseline (speedup 1.0000x reference)
import functools

import jax
import jax.numpy as jnp
from jax import lax
from jax.experimental import pallas as pl
from jax.experimental.pallas import tpu as pltpu
from jax.experimental.pallas import tpu_sc as plsc

MAXLEN = 200
EMBED = 64
WIDE = 128
BATCH = 1024
SEQ = 200

NC, NS, LANES = 2, 16, 16
NW = NC * NS
ROWS = BATCH * SEQ
RPW = ROWS // NW
CHUNK = 128
NCHUNK = RPW // CHUNK


def _sc_embed(x2, tt_pad, pos_table):
    mesh = plsc.VectorSubcoreMesh(core_axis_name="c", subcore_axis_name="s")

    @functools.partial(
        pl.kernel,
        out_type=jax.ShapeDtypeStruct((ROWS, WIDE), jnp.float32),
        mesh=mesh,
        scratch_types=[
            pltpu.VMEM((CHUNK,), jnp.int32),
            pltpu.VMEM((CHUNK, WIDE), jnp.float32),
            pltpu.VMEM((MAXLEN, EMBED), jnp.float32),
            pltpu.SemaphoreType.DMA,
        ],
    )
    def k(x_hbm, tab_hbm, pos_hbm, out_hbm, idx_v, rows_v, pos_v, sem):
        wid = lax.axis_index("s") * NC + lax.axis_index("c")
        pltpu.sync_copy(pos_hbm, pos_v)

        @pl.loop(0, NCHUNK)
        def _(kc):
            g = wid * NCHUNK + kc
            base = g * CHUNK
            pltpu.sync_copy(x_hbm.at[g], idx_v)
            pltpu.async_copy(tab_hbm.at[idx_v], rows_v, sem).wait()

            p0 = lax.rem(base, MAXLEN)
            seg1 = jnp.minimum(CHUNK, MAXLEN - p0)

            @pl.loop(0, seg1)
            def _(r):
                for c in range(0, EMBED, LANES):
                    rows_v[r, pl.ds(c, LANES)] = (
                        rows_v[r, pl.ds(c, LANES)] + pos_v[p0 + r, pl.ds(c, LANES)]
                    )

            @pl.loop(seg1, CHUNK)
            def _(r):
                for c in range(0, EMBED, LANES):
                    rows_v[r, pl.ds(c, LANES)] = (
                        rows_v[r, pl.ds(c, LANES)]
                        + pos_v[p0 + r - MAXLEN, pl.ds(c, LANES)]
                    )

            pltpu.sync_copy(rows_v, out_hbm.at[pl.ds(base, CHUNK)])

    return k(x2, tt_pad, pos_table)


def kernel(x, token_table, pos_table):
    x2 = x.reshape(ROWS // CHUNK, CHUNK).astype(jnp.int32)
    tt_pad = jnp.pad(token_table, ((0, 0), (0, WIDE - EMBED)))
    out = _sc_embed(x2, tt_pad, pos_table)
    return out[:, :EMBED].reshape(BATCH, SEQ, EMBED)
